# baseline (speedup 1.0000x reference)
import functools

import jax
import jax.numpy as jnp
from jax import lax
from jax.experimental import pallas as pl
from jax.experimental.pallas import tpu as pltpu
from jax.experimental.pallas import tpu_sc as plsc

_N, _E, _F, _D, _C = 10000, 320000, 128, 32, 2
_BN_EPS = 1e-5
_NC, _NS = 2, 16
_RPT = _N // _NS
_CHUNK = 112
_SLAB = 180
_PAD = _NS * _SLAB * _CHUNK - _E

_FAST = 1
_CF1, _CS1 = 90, 90
_CF2, _CS2 = 90, 90


def _sc_segment_sum(h, srcp, dstp, zer, width, cpt_f, cpt_s):
    mesh = plsc.VectorSubcoreMesh(core_axis_name="c", subcore_axis_name="s")

    @functools.partial(
        pl.kernel,
        mesh=mesh,
        compiler_params=pltpu.CompilerParams(use_tc_tiling_on_sc=False),
        out_type=jax.ShapeDtypeStruct((_NC * _N, _F), jnp.float32),
        scratch_types=[
            pltpu.VMEM((cpt_f, _CHUNK), jnp.int32),
            pltpu.VMEM((cpt_f, _CHUNK), jnp.int32),
            pltpu.VMEM((_CHUNK, width), jnp.float32),
            pltpu.VMEM((_CHUNK, width), jnp.float32),
            pltpu.VMEM_SHARED((_N, width), jnp.float32),
            pltpu.SemaphoreType.DMA,
            pltpu.SemaphoreType.DMA,
        ],
    )
    def k(h_hbm, srcp_hbm, dstp_hbm, zer_hbm, out_hbm,
          src_v, dst_v, rows0, rows1, acc_sh, sem0, sem1):
        c = lax.axis_index("c")
        s = lax.axis_index("s")
        fast = c == _FAST
        mycpt = jnp.where(fast, cpt_f, cpt_s)
        r0 = s * _RPT
        pltpu.sync_copy(zer_hbm, acc_sh.at[pl.ds(r0, _RPT)])

        @pl.when(fast)
        def _():
            pltpu.sync_copy(srcp_hbm.at[s, pl.ds(0, cpt_f)], src_v)
            pltpu.sync_copy(dstp_hbm.at[s, pl.ds(0, cpt_f)], dst_v)

        @pl.when(jnp.logical_not(fast))
        def _():
            pltpu.sync_copy(srcp_hbm.at[s, pl.ds(cpt_f, cpt_s)],
                            src_v.at[pl.ds(0, cpt_s)])
            pltpu.sync_copy(dstp_hbm.at[s, pl.ds(cpt_f, cpt_s)],
                            dst_v.at[pl.ds(0, cpt_s)])

        plsc.subcore_barrier()

        bufs = (rows0, rows1)
        sems = (sem0, sem1)
        for b in range(2):
            pltpu.async_copy(h_hbm.at[src_v.at[b]], bufs[b], sems[b])

        def body(i, carry):
            j = 2 * i
            for b in range(2):
                pltpu.make_async_copy(h_hbm.at[src_v.at[0]],
                                      bufs[b], sems[b]).wait()
                pltpu.sync_copy(bufs[b], acc_sh.at[dst_v.at[j + b]], add=True)
                pltpu.async_copy(h_hbm.at[src_v.at[j + 2 + b]],
                                 bufs[b], sems[b])
            return carry

        lax.fori_loop(0, mycpt // 2 - 1, body, 0)
        for b in range(2):
            pltpu.make_async_copy(h_hbm.at[src_v.at[0]],
                                  bufs[b], sems[b]).wait()
            pltpu.sync_copy(bufs[b], acc_sh.at[dst_v.at[mycpt - 2 + b]],
                            add=True)
        plsc.subcore_barrier()
        pltpu.sync_copy(acc_sh.at[pl.ds(r0, _RPT)],
                        out_hbm.at[pl.ds(c * _N + r0, _RPT), pl.ds(0, width)])

    return k(h, srcp, dstp, zer)


_GRID = 5
_BR = _N // _GRID

def _row_spec(width):
    return pl.BlockSpec((_BR, width), lambda i: (i, 0))

def _p1_spec(width):
    return pl.BlockSpec((_BR, width), lambda i: (i + _GRID, 0))

def _full_spec(a, b):
    return pl.BlockSpec((a, b), lambda i: (0, 0))

_INVSQ = 1.0 / (1.0 + _BN_EPS) ** 0.5


def _tc_layer(p, h, Wa, ba, Wb, bb, g, be, width):
    def body(p0_ref, p1_ref, h_ref, wa_ref, ba_ref, wb_ref, bb_ref, g_ref,
             be_ref, o_ref):
        u = h_ref[...] + p0_ref[:, :width] + p1_ref[:, :width]
        row = lax.broadcasted_iota(jnp.int32, (_BR, 1), 0) + pl.program_id(0) * _BR
        u = u - jnp.where(row == 0, jnp.float32(_PAD), 0.0) * h_ref[...]
        t1 = jax.nn.relu(jnp.dot(u, wa_ref[...],
                                 preferred_element_type=jnp.float32)
                         + ba_ref[...])
        t = jnp.dot(t1, wb_ref[...],
                    preferred_element_type=jnp.float32) + bb_ref[...]
        o_ref[...] = jax.nn.relu(t) * (g_ref[...] * _INVSQ) + be_ref[...]
    return pl.pallas_call(
        body,
        grid=(_GRID,),
        in_specs=[_row_spec(_F), _p1_spec(_F), _row_spec(width),
                  _full_spec(width, _D), _full_spec(1, _D),
                  _full_spec(_D, _D), _full_spec(1, _D), _full_spec(1, _D),
                  _full_spec(1, _D)],
        out_specs=_row_spec(_D),
        out_shape=jax.ShapeDtypeStruct((_N, _D), jnp.float32),
    )(p, p, h, Wa, ba, Wb, bb, g, be)


def _tc_head(p, h, Wa, ba, Wb, bb, g, be, Wf1, bf1, Wf2, bf2):
    def body(p0_ref, p1_ref, h_ref, wa_ref, ba_ref, wb_ref, bb_ref, g_ref,
             be_ref, wf1_ref, bf1_ref, wf2_ref, bf2_ref, o_ref):
        u = h_ref[...] + p0_ref[:, :_D] + p1_ref[:, :_D]
        row = lax.broadcasted_iota(jnp.int32, (_BR, 1), 0) + pl.program_id(0) * _BR
        u = u - jnp.where(row == 0, jnp.float32(_PAD), 0.0) * h_ref[...]
        t1 = jax.nn.relu(jnp.dot(u, wa_ref[...],
                                 preferred_element_type=jnp.float32)
                         + ba_ref[...])
        t = jnp.dot(t1, wb_ref[...],
                    preferred_element_type=jnp.float32) + bb_ref[...]
        hh = jax.nn.relu(t) * (g_ref[...] * _INVSQ) + be_ref[...]
        hh = jax.nn.relu(jnp.dot(hh, wf1_ref[...],
                                 preferred_element_type=jnp.float32)
                         + bf1_ref[...])
        o_ref[...] = jnp.dot(hh, wf2_ref[...],
                             preferred_element_type=jnp.float32) + bf2_ref[...]
    return pl.pallas_call(
        body,
        grid=(_GRID,),
        in_specs=[_row_spec(_F), _p1_spec(_F), _row_spec(_D),
                  _full_spec(_D, _D), _full_spec(1, _D), _full_spec(_D, _D),
                  _full_spec(1, _D), _full_spec(1, _D), _full_spec(1, _D),
                  _full_spec(_D, _D), _full_spec(1, _D), _full_spec(_D, _C),
                  _full_spec(1, _C)],
        out_specs=_row_spec(_C),
        out_shape=jax.ShapeDtypeStruct((_N, _C), jnp.float32),
    )(p, p, h, Wa, ba, Wb, bb, g, be, Wf1, bf1, Wf2, bf2)


def kernel(x, edge_index, edge_attr, batch,
           W11, b11, W12, b12, g1, be1,
           W21, b21, W22, b22, g2, be2,
           W31, b31, W32, b32, g3, be3,
           Wf1, bf1, Wf2, bf2):
    pad = jnp.zeros((2, _PAD), jnp.int32)
    ep = jnp.concatenate([edge_index, pad], axis=1)
    srcp = ep[0].reshape(_NS, _SLAB, _CHUNK)
    dstp = ep[1].reshape(_NS, _SLAB, _CHUNK)
    zerF = jnp.zeros((_RPT, _F), jnp.float32)
    zerD = jnp.zeros((_RPT, _D), jnp.float32)

    r = lambda v: v.reshape(1, -1)

    p1 = _sc_segment_sum(x, srcp, dstp, zerF, _F, _CF1, _CS1)
    h1 = _tc_layer(p1, x, W11, r(b11), W12, r(b12), r(g1), r(be1), _F)
    p2 = _sc_segment_sum(h1, srcp, dstp, zerD, _D, _CF2, _CS2)
    h2 = _tc_layer(p2, h1, W21, r(b21), W22, r(b22), r(g2), r(be2), _D)
    p3 = _sc_segment_sum(h2, srcp, dstp, zerD, _D, _CF2, _CS2)
    out = _tc_head(p3, h2, W31, r(b31), W32, r(b32), r(g3), r(be3),
                   Wf1, r(bf1), Wf2, r(bf2))
    return out


# speedup vs baseline: 1.3459x; 1.3459x over previous
import functools

import jax
import jax.numpy as jnp
from jax import lax
from jax.experimental import pallas as pl
from jax.experimental.pallas import tpu as pltpu
from jax.experimental.pallas import tpu_sc as plsc

_N, _E, _F, _D, _C = 10000, 320000, 128, 32, 2
_BN_EPS = 1e-5
_NC, _NS = 2, 16
_RPT = _N // _NS
_CHUNK = 80
_SLAB = 250

_FAST = 1
_CF1, _CS1 = 126, 124
_CF2, _CS2 = 126, 124


def _sc_segment_sum(h, srcp, dstp, zer, width, cpt_f, cpt_s):
    mesh = plsc.VectorSubcoreMesh(core_axis_name="c", subcore_axis_name="s")

    @functools.partial(
        pl.kernel,
        mesh=mesh,
        compiler_params=pltpu.CompilerParams(use_tc_tiling_on_sc=False),
        out_type=jax.ShapeDtypeStruct((_NC * _N, _F), jnp.float32),
        scratch_types=[
            pltpu.VMEM((cpt_f, _CHUNK), jnp.int32),
            pltpu.VMEM((cpt_f, _CHUNK), jnp.int32),
            pltpu.VMEM((_CHUNK, width), jnp.float32),
            pltpu.VMEM((_CHUNK, width), jnp.float32),
            pltpu.VMEM_SHARED((_N, width), jnp.float32),
            pltpu.SemaphoreType.DMA,
            pltpu.SemaphoreType.DMA,
        ],
    )
    def k(h_hbm, srcp_hbm, dstp_hbm, zer_hbm, out_hbm,
          src_v, dst_v, rows0, rows1, acc_sh, sem0, sem1):
        c = lax.axis_index("c")
        s = lax.axis_index("s")
        fast = c == _FAST
        mycpt = jnp.where(fast, cpt_f, cpt_s)
        r0 = s * _RPT
        pltpu.sync_copy(zer_hbm, acc_sh.at[pl.ds(r0, _RPT)])

        @pl.when(fast)
        def _():
            pltpu.sync_copy(srcp_hbm.at[s, pl.ds(0, cpt_f)], src_v)
            pltpu.sync_copy(dstp_hbm.at[s, pl.ds(0, cpt_f)], dst_v)

        @pl.when(jnp.logical_not(fast))
        def _():
            pltpu.sync_copy(srcp_hbm.at[s, pl.ds(cpt_f, cpt_s)],
                            src_v.at[pl.ds(0, cpt_s)])
            pltpu.sync_copy(dstp_hbm.at[s, pl.ds(cpt_f, cpt_s)],
                            dst_v.at[pl.ds(0, cpt_s)])

        plsc.subcore_barrier()

        bufs = (rows0, rows1)
        sems = (sem0, sem1)
        for b in range(2):
            pltpu.async_copy(h_hbm.at[src_v.at[b]], bufs[b], sems[b])

        def body(i, carry):
            j = 2 * i
            for b in range(2):
                pltpu.make_async_copy(h_hbm.at[src_v.at[0]],
                                      bufs[b], sems[b]).wait()
                pltpu.sync_copy(bufs[b], acc_sh.at[dst_v.at[j + b]], add=True)
                pltpu.async_copy(h_hbm.at[src_v.at[j + 2 + b]],
                                 bufs[b], sems[b])
            return carry

        lax.fori_loop(0, mycpt // 2 - 1, body, 0)
        for b in range(2):
            pltpu.make_async_copy(h_hbm.at[src_v.at[0]],
                                  bufs[b], sems[b]).wait()
            pltpu.sync_copy(bufs[b], acc_sh.at[dst_v.at[mycpt - 2 + b]],
                            add=True)
        plsc.subcore_barrier()
        pltpu.sync_copy(acc_sh.at[pl.ds(r0, _RPT)],
                        out_hbm.at[pl.ds(c * _N + r0, _RPT), pl.ds(0, width)])

    return k(h, srcp, dstp, zer)


_GRID = 5
_BR = _N // _GRID

def _row_spec(width):
    return pl.BlockSpec((_BR, width), lambda i: (i, 0))

def _p1_spec(width):
    return pl.BlockSpec((_BR, width), lambda i: (i + _GRID, 0))

def _full_spec(a, b):
    return pl.BlockSpec((a, b), lambda i: (0, 0))

_INVSQ = 1.0 / (1.0 + _BN_EPS) ** 0.5


def _tc_layer(p, h, Wa, ba, Wb, bb, g, be, width):
    def body(p0_ref, p1_ref, h_ref, wa_ref, ba_ref, wb_ref, bb_ref, g_ref,
             be_ref, o_ref):
        u = h_ref[...] + p0_ref[:, :width] + p1_ref[:, :width]
        t1 = jax.nn.relu(jnp.dot(u, wa_ref[...],
                                 preferred_element_type=jnp.float32)
                         + ba_ref[...])
        t = jnp.dot(t1, wb_ref[...],
                    preferred_element_type=jnp.float32) + bb_ref[...]
        o_ref[...] = jax.nn.relu(t) * (g_ref[...] * _INVSQ) + be_ref[...]
    return pl.pallas_call(
        body,
        grid=(_GRID,),
        in_specs=[_row_spec(_F), _p1_spec(_F), _row_spec(width),
                  _full_spec(width, _D), _full_spec(1, _D),
                  _full_spec(_D, _D), _full_spec(1, _D), _full_spec(1, _D),
                  _full_spec(1, _D)],
        out_specs=_row_spec(_D),
        out_shape=jax.ShapeDtypeStruct((_N, _D), jnp.float32),
    )(p, p, h, Wa, ba, Wb, bb, g, be)


def _tc_head(p, h, Wa, ba, Wb, bb, g, be, Wf1, bf1, Wf2, bf2):
    def body(p0_ref, p1_ref, h_ref, wa_ref, ba_ref, wb_ref, bb_ref, g_ref,
             be_ref, wf1_ref, bf1_ref, wf2_ref, bf2_ref, o_ref):
        u = h_ref[...] + p0_ref[:, :_D] + p1_ref[:, :_D]
        t1 = jax.nn.relu(jnp.dot(u, wa_ref[...],
                                 preferred_element_type=jnp.float32)
                         + ba_ref[...])
        t = jnp.dot(t1, wb_ref[...],
                    preferred_element_type=jnp.float32) + bb_ref[...]
        hh = jax.nn.relu(t) * (g_ref[...] * _INVSQ) + be_ref[...]
        hh = jax.nn.relu(jnp.dot(hh, wf1_ref[...],
                                 preferred_element_type=jnp.float32)
                         + bf1_ref[...])
        o_ref[...] = jnp.dot(hh, wf2_ref[...],
                             preferred_element_type=jnp.float32) + bf2_ref[...]
    return pl.pallas_call(
        body,
        grid=(_GRID,),
        in_specs=[_row_spec(_F), _p1_spec(_F), _row_spec(_D),
                  _full_spec(_D, _D), _full_spec(1, _D), _full_spec(_D, _D),
                  _full_spec(1, _D), _full_spec(1, _D), _full_spec(1, _D),
                  _full_spec(_D, _D), _full_spec(1, _D), _full_spec(_D, _C),
                  _full_spec(1, _C)],
        out_specs=_row_spec(_C),
        out_shape=jax.ShapeDtypeStruct((_N, _C), jnp.float32),
    )(p, p, h, Wa, ba, Wb, bb, g, be, Wf1, bf1, Wf2, bf2)


def kernel(x, edge_index, edge_attr, batch,
           W11, b11, W12, b12, g1, be1,
           W21, b21, W22, b22, g2, be2,
           W31, b31, W32, b32, g3, be3,
           Wf1, bf1, Wf2, bf2):
    srcp = edge_index[0].reshape(_NS, _SLAB, _CHUNK)
    dstp = edge_index[1].reshape(_NS, _SLAB, _CHUNK)
    zerF = jnp.zeros((_RPT, _F), jnp.float32)
    zerD = jnp.zeros((_RPT, _D), jnp.float32)

    r = lambda v: v.reshape(1, -1)

    p1 = _sc_segment_sum(x, srcp, dstp, zerF, _F, _CF1, _CS1)
    h1 = _tc_layer(p1, x, W11, r(b11), W12, r(b12), r(g1), r(be1), _F)
    p2 = _sc_segment_sum(h1, srcp, dstp, zerD, _D, _CF2, _CS2)
    h2 = _tc_layer(p2, h1, W21, r(b21), W22, r(b22), r(g2), r(be2), _D)
    p3 = _sc_segment_sum(h2, srcp, dstp, zerD, _D, _CF2, _CS2)
    out = _tc_head(p3, h2, W31, r(b31), W32, r(b32), r(g3), r(be3),
                   Wf1, r(bf1), Wf2, r(bf2))
    return out
